# baseline (speedup 1.0000x reference)
import functools

import jax
import jax.numpy as jnp
from jax import lax
from jax.experimental import pallas as pl
from jax.experimental.pallas import tpu as pltpu
from jax.experimental.pallas import tpu_sc as plsc

NC = 2
NS = 16
NW = NC * NS

EMB = 64
IDX_MINOR = 128
CHUNK = 512
K = CHUNK // IDX_MINOR


def _emb_body(idx_hbm, table_hbm, out_hbm, idx_v, rows0, rows1, gsem0, gsem1,
              osem0, osem1, *, rows_per_w, groups_per_w):
    chunks = rows_per_w // CHUNK
    wid = lax.axis_index("s") * NC + lax.axis_index("c")
    base = wid * rows_per_w

    pltpu.sync_copy(idx_hbm.at[wid], idx_v)

    rows = (rows0, rows1)
    gsems = (gsem0, gsem1)
    osems = (osem0, osem1)

    def gather_descr(g, b, j):
        return pltpu.make_async_copy(
            table_hbm.at[idx_v.at[g * K + j]],
            rows[b].at[pl.ds(j * IDX_MINOR, IDX_MINOR)],
            gsems[b],
        )

    def out_descr(g, b):
        return pltpu.make_async_copy(
            rows[b],
            out_hbm.at[pl.ds(base + g * CHUNK, CHUNK)],
            osems[b],
        )

    def start_gathers(g, b):
        for j in range(K):
            gather_descr(g, b, j).start()

    def wait_gathers(g, b):
        for j in range(K):
            gather_descr(g, b, j).wait()

    start_gathers(0, 0)
    start_gathers(1, 1)

    def loop_body(i, carry):
        del carry
        for b in range(2):
            g = 2 * i + b
            wait_gathers(g, b)
            out_descr(g, b).start()
            nxt = g + 2

            @pl.when(nxt < chunks)
            def _():
                out_descr(g, b).wait()
                start_gathers(nxt, b)

        return 0

    lax.fori_loop(0, chunks // 2, loop_body, 0)

    for b in range(2):
        out_descr(chunks - 2 + b, b).wait()


def _emb_lookup(idx_grouped, table, rows_per_w, groups_per_w):
    total = NW * rows_per_w
    mesh = plsc.VectorSubcoreMesh(core_axis_name="c", subcore_axis_name="s")
    body = functools.partial(
        _emb_body, rows_per_w=rows_per_w, groups_per_w=groups_per_w)
    return pl.kernel(
        body,
        out_type=jax.ShapeDtypeStruct((total, EMB), jnp.float32),
        mesh=mesh,
        compiler_params=pltpu.CompilerParams(use_tc_tiling_on_sc=False),
        scratch_types=[
            pltpu.VMEM((groups_per_w, IDX_MINOR), jnp.int32),
            pltpu.VMEM((CHUNK, EMB), jnp.float32),
            pltpu.VMEM((CHUNK, EMB), jnp.float32),
            pltpu.SemaphoreType.DMA,
            pltpu.SemaphoreType.DMA,
            pltpu.SemaphoreType.DMA,
            pltpu.SemaphoreType.DMA,
        ],
    )(idx_grouped, table)


def kernel(x, table):
    batch, seq = x.shape
    total = batch * seq
    rows_per_w = total // NW
    groups_per_w = rows_per_w // IDX_MINOR
    idx_grouped = jnp.reshape(x.astype(jnp.int32), (NW, groups_per_w, IDX_MINOR))
    out = _emb_lookup(idx_grouped, table, rows_per_w, groups_per_w)
    return jnp.reshape(out, (batch, seq, EMB))
